# baseline (speedup 1.0000x reference)
import functools

import jax
import jax.numpy as jnp
from jax import lax
from jax.experimental import pallas as pl
from jax.experimental.pallas import tpu as pltpu
from jax.experimental.pallas import tpu_sc as plsc

B = 16384
SED = 32
PED = 64
NUM_MODES = 12
LINE = 128


def _sc_gather(table_u, idx2):
    NW, b_per_w = idx2.shape
    mesh = plsc.VectorSubcoreMesh(core_axis_name="c", subcore_axis_name="s")
    nc = mesh.num_cores

    @functools.partial(
        pl.kernel,
        out_type=jax.ShapeDtypeStruct((NW * b_per_w, LINE), jnp.uint32),
        mesh=mesh,
        scratch_types=[
            pltpu.VMEM((b_per_w,), jnp.int32),
            pltpu.VMEM((b_per_w, LINE), jnp.uint32),
            pltpu.SemaphoreType.DMA,
        ],
    )
    def gather_kernel(table_hbm, idx_hbm, out_hbm, idx_v, rows_v, sem):
        wid = lax.axis_index("s") * nc + lax.axis_index("c")
        base = wid * b_per_w
        pltpu.sync_copy(idx_hbm.at[wid], idx_v)

        def body(g, carry):
            v = idx_v[pl.ds(g * 16, 16)]
            for l in range(16):
                r = v[l]
                pltpu.async_copy(
                    table_hbm.at[pl.ds(r, 1)],
                    rows_v.at[pl.ds(g * 16 + l, 1)],
                    sem,
                )
            return carry

        lax.fori_loop(0, b_per_w // 16, body, 0)
        pltpu.make_async_copy(
            table_hbm.at[pl.ds(0, b_per_w)], rows_v, sem
        ).wait()
        pltpu.sync_copy(rows_v, out_hbm.at[pl.ds(base, b_per_w)])

    return gather_kernel(table_u, idx2)


def _tc_fused_t(lines, uid2d, tmT, tsT, mode_table, W_time, b_time2d,
                wu_perm, wmt, b_pref2d):
    bs = 2048
    grid = (B // bs,)
    half = SED // 2

    def body(u_ref, uid_ref, tm_ref, ts_ref, mt_ref, wt_ref, bt_ref,
             wup_ref, wmt_ref, bp_ref, o_ref):
        u32b = u_ref[...]
        lo = lax.bitcast_convert_type(u32b << 16, jnp.float32)
        hi = lax.bitcast_convert_type(
            u32b & jnp.uint32(0xFFFF0000), jnp.float32
        )
        off = uid_ref[...] & 7
        acc_lo = jnp.zeros((bs, half), jnp.float32)
        acc_hi = jnp.zeros((bs, half), jnp.float32)
        for o in range(8):
            m = off == o
            acc_lo = acc_lo + jnp.where(m, lo[:, o * half : (o + 1) * half], 0.0)
            acc_hi = acc_hi + jnp.where(m, hi[:, o * half : (o + 1) * half], 0.0)
        u_cat = jnp.concatenate([acc_lo, acc_hi], axis=1)
        ts = ts_ref[...]
        tm = tm_ref[...]
        wmt_b = wmt_ref[...]
        time_embT = lax.dot_general(
            wt_ref[...], ts, (((0,), (0,)), ((), ())),
            preferred_element_type=jnp.float32,
        ) + bt_ref[...]
        onehotT = (
            lax.broadcasted_iota(jnp.int32, (NUM_MODES, bs), 0) == tm
        ).astype(jnp.float32)
        mode_embT = lax.dot_general(
            mt_ref[...], onehotT, (((0,), (0,)), ((), ())),
            preferred_element_type=jnp.float32,
        )
        acc = lax.dot_general(
            wup_ref[...], u_cat, (((1,), (1,)), ((), ())),
            preferred_element_type=jnp.float32,
        )
        acc = acc + lax.dot_general(
            wmt_b[:, 0:SED], mode_embT, (((1,), (0,)), ((), ())),
            preferred_element_type=jnp.float32,
        )
        acc = acc + lax.dot_general(
            wmt_b[:, SED:], time_embT, (((1,), (0,)), ((), ())),
            preferred_element_type=jnp.float32,
        )
        o_ref[...] = acc + bp_ref[...]

    return pl.pallas_call(
        body,
        grid=grid,
        in_specs=[
            pl.BlockSpec((bs, LINE), lambda i: (i, 0)),
            pl.BlockSpec((bs, 1), lambda i: (i, 0)),
            pl.BlockSpec((1, bs), lambda i: (0, i)),
            pl.BlockSpec((6, bs), lambda i: (0, i)),
            pl.BlockSpec((NUM_MODES, SED), lambda i: (0, 0)),
            pl.BlockSpec((6, SED), lambda i: (0, 0)),
            pl.BlockSpec((SED, 1), lambda i: (0, 0)),
            pl.BlockSpec((PED, SED), lambda i: (0, 0)),
            pl.BlockSpec((PED, 2 * SED), lambda i: (0, 0)),
            pl.BlockSpec((PED, 1), lambda i: (0, 0)),
        ],
        out_specs=pl.BlockSpec((PED, bs), lambda i: (0, i)),
        out_shape=jax.ShapeDtypeStruct((PED, B), jnp.float32),
    )(lines, uid2d, tmT, tsT, mode_table, W_time, b_time2d, wu_perm, wmt,
      b_pref2d)


def kernel(user_id, transport_mode, timestamp, user_table, mode_table,
           W_time, b_time, W_pref, b_pref):
    info = plsc.get_sparse_core_info()
    NW = info.num_cores * info.num_subcores
    uid = user_id.astype(jnp.int32)
    n_rows = user_table.shape[0]
    bf = user_table.astype(jnp.bfloat16)
    table_u = lax.bitcast_convert_type(
        bf.reshape(n_rows, SED // 2, 2), jnp.uint32
    ).reshape(n_rows * SED // (2 * LINE), LINE)
    idx2 = (uid >> 3).reshape(NW, B // NW)
    lines = _sc_gather(table_u, idx2)
    WpT = W_pref.T
    perm = jnp.array(
        [2 * e for e in range(SED // 2)] + [2 * e + 1 for e in range(SED // 2)],
        dtype=jnp.int32,
    )
    wu_perm = WpT[:, 0:SED][:, perm]
    wmt = WpT[:, SED:]
    outT = _tc_fused_t(
        lines,
        uid.reshape(B, 1),
        transport_mode.astype(jnp.int32).reshape(1, B),
        timestamp.T,
        mode_table,
        W_time,
        b_time.reshape(SED, 1),
        wu_perm,
        wmt,
        b_pref2d=b_pref.reshape(PED, 1),
    )
    return outT.T


# speedup vs baseline: 3.4277x; 3.4277x over previous
import functools

import jax
import jax.numpy as jnp
from jax import lax
from jax.experimental import pallas as pl
from jax.experimental.pallas import tpu as pltpu
from jax.experimental.pallas import tpu_sc as plsc

B = 16384
SED = 32
PED = 64
NUM_MODES = 12


def _sc_gather(user_table, idx2):
    NW, b_per_w = idx2.shape
    mesh = plsc.VectorSubcoreMesh(core_axis_name="c", subcore_axis_name="s")
    nc = mesh.num_cores

    @functools.partial(
        pl.kernel,
        out_type=jax.ShapeDtypeStruct((NW * b_per_w, SED), jnp.float32),
        mesh=mesh,
        scratch_types=[
            pltpu.VMEM((b_per_w,), jnp.int32),
            pltpu.VMEM((b_per_w, SED), jnp.float32),
            pltpu.SemaphoreType.DMA,
        ],
    )
    def gather_kernel(table_hbm, idx_hbm, out_hbm, idx_v, rows_v, sem):
        wid = lax.axis_index("s") * nc + lax.axis_index("c")
        base = wid * b_per_w
        pltpu.sync_copy(idx_hbm.at[wid], idx_v)

        def body(g, carry):
            v = idx_v[pl.ds(g * 16, 16)]
            for l in range(16):
                r = v[l]
                pltpu.async_copy(
                    table_hbm.at[pl.ds(r, 1)],
                    rows_v.at[pl.ds(g * 16 + l, 1)],
                    sem,
                )
            return carry

        lax.fori_loop(0, b_per_w // 16, body, 0)
        pltpu.make_async_copy(
            table_hbm.at[pl.ds(0, b_per_w)], rows_v, sem
        ).wait()
        pltpu.sync_copy(rows_v, out_hbm.at[pl.ds(base, b_per_w)])

    return gather_kernel(user_table, idx2)


def _tc_fused_t(rows, tmT, tsT, mode_table, W_time, b_time2d, WpT, b_pref2d):
    bs = 2048
    grid = (B // bs,)

    def body(u_ref, tm_ref, ts_ref, mt_ref, wt_ref, bt_ref, wpt_ref, bp_ref,
             o_ref):
        u = u_ref[...]
        ts = ts_ref[...]
        tm = tm_ref[...]
        wpt = wpt_ref[...]
        time_embT = lax.dot_general(
            wt_ref[...], ts, (((0,), (0,)), ((), ())),
            preferred_element_type=jnp.float32,
        ) + bt_ref[...]
        onehotT = (
            lax.broadcasted_iota(jnp.int32, (NUM_MODES, bs), 0) == tm
        ).astype(jnp.float32)
        mode_embT = lax.dot_general(
            mt_ref[...], onehotT, (((0,), (0,)), ((), ())),
            preferred_element_type=jnp.float32,
        )
        acc = lax.dot_general(
            wpt[:, 0:SED], u, (((1,), (1,)), ((), ())),
            preferred_element_type=jnp.float32,
        )
        acc = acc + lax.dot_general(
            wpt[:, SED : 2 * SED], mode_embT, (((1,), (0,)), ((), ())),
            preferred_element_type=jnp.float32,
        )
        acc = acc + lax.dot_general(
            wpt[:, 2 * SED :], time_embT, (((1,), (0,)), ((), ())),
            preferred_element_type=jnp.float32,
        )
        o_ref[...] = acc + bp_ref[...]

    return pl.pallas_call(
        body,
        grid=grid,
        in_specs=[
            pl.BlockSpec((bs, SED), lambda i: (i, 0)),
            pl.BlockSpec((1, bs), lambda i: (0, i)),
            pl.BlockSpec((6, bs), lambda i: (0, i)),
            pl.BlockSpec((NUM_MODES, SED), lambda i: (0, 0)),
            pl.BlockSpec((6, SED), lambda i: (0, 0)),
            pl.BlockSpec((SED, 1), lambda i: (0, 0)),
            pl.BlockSpec((PED, 3 * SED), lambda i: (0, 0)),
            pl.BlockSpec((PED, 1), lambda i: (0, 0)),
        ],
        out_specs=pl.BlockSpec((PED, bs), lambda i: (0, i)),
        out_shape=jax.ShapeDtypeStruct((PED, B), jnp.float32),
    )(rows, tmT, tsT, mode_table, W_time, b_time2d, WpT, b_pref2d)


def kernel(user_id, transport_mode, timestamp, user_table, mode_table,
           W_time, b_time, W_pref, b_pref):
    info = plsc.get_sparse_core_info()
    NW = info.num_cores * info.num_subcores
    uid = user_id.astype(jnp.int32)
    idx2 = uid.reshape(NW, B // NW)
    rows = _sc_gather(user_table, idx2)
    outT = _tc_fused_t(
        rows,
        transport_mode.astype(jnp.int32).reshape(1, B),
        timestamp.T,
        mode_table,
        W_time,
        b_time.reshape(SED, 1),
        W_pref.T,
        b_pref.reshape(PED, 1),
    )
    return outT.T


# speedup vs baseline: 3.4521x; 1.0071x over previous
import functools

import jax
import jax.numpy as jnp
from jax import lax
from jax.experimental import pallas as pl
from jax.experimental.pallas import tpu as pltpu
from jax.experimental.pallas import tpu_sc as plsc

B = 16384
SED = 32
PED = 64
NUM_MODES = 12


def _sc_gather(user_table, idx2):
    NW, b_per_w = idx2.shape
    mesh = plsc.VectorSubcoreMesh(core_axis_name="c", subcore_axis_name="s")
    nc = mesh.num_cores

    @functools.partial(
        pl.kernel,
        out_type=jax.ShapeDtypeStruct((NW * b_per_w, SED), jnp.float32),
        mesh=mesh,
        scratch_types=[
            pltpu.VMEM((b_per_w,), jnp.int32),
            pltpu.VMEM((b_per_w, SED), jnp.float32),
            pltpu.SemaphoreType.DMA,
        ],
    )
    def gather_kernel(table_hbm, idx_hbm, out_hbm, idx_v, rows_v, sem):
        wid = lax.axis_index("s") * nc + lax.axis_index("c")
        base = wid * b_per_w
        pltpu.sync_copy(idx_hbm.at[wid], idx_v)

        def body(g, carry):
            v = idx_v[pl.ds(g * 16, 16)]
            for l in range(16):
                r = v[l]
                pltpu.async_copy(
                    table_hbm.at[pl.ds(r, 1)],
                    rows_v.at[pl.ds(g * 16 + l, 1)],
                    sem,
                )
            return carry

        lax.fori_loop(0, b_per_w // 16, body, 0)
        pltpu.make_async_copy(
            table_hbm.at[pl.ds(0, b_per_w)], rows_v, sem
        ).wait()
        pltpu.sync_copy(rows_v, out_hbm.at[pl.ds(base, b_per_w)])

    return gather_kernel(user_table, idx2)


def _tc_fused_t(rows, tmT, tsT, mode_table, W_time, b_time2d, WpT, b_pref2d):
    bs = 4096
    grid = (B // bs,)

    def body(u_ref, tm_ref, ts_ref, mt_ref, wt_ref, bt_ref, wpt_ref, bp_ref,
             o_ref):
        u = u_ref[...]
        ts = ts_ref[...]
        tm = tm_ref[...]
        wpt = wpt_ref[...]
        time_embT = lax.dot_general(
            wt_ref[...], ts, (((0,), (0,)), ((), ())),
            preferred_element_type=jnp.float32,
        ) + bt_ref[...]
        onehotT = (
            lax.broadcasted_iota(jnp.int32, (NUM_MODES, bs), 0) == tm
        ).astype(jnp.float32)
        mode_embT = lax.dot_general(
            mt_ref[...], onehotT, (((0,), (0,)), ((), ())),
            preferred_element_type=jnp.float32,
        )
        acc = lax.dot_general(
            wpt[:, 0:SED], u, (((1,), (1,)), ((), ())),
            preferred_element_type=jnp.float32,
        )
        acc = acc + lax.dot_general(
            wpt[:, SED : 2 * SED], mode_embT, (((1,), (0,)), ((), ())),
            preferred_element_type=jnp.float32,
        )
        acc = acc + lax.dot_general(
            wpt[:, 2 * SED :], time_embT, (((1,), (0,)), ((), ())),
            preferred_element_type=jnp.float32,
        )
        o_ref[...] = acc + bp_ref[...]

    return pl.pallas_call(
        body,
        grid=grid,
        in_specs=[
            pl.BlockSpec((bs, SED), lambda i: (i, 0)),
            pl.BlockSpec((1, bs), lambda i: (0, i)),
            pl.BlockSpec((6, bs), lambda i: (0, i)),
            pl.BlockSpec((NUM_MODES, SED), lambda i: (0, 0)),
            pl.BlockSpec((6, SED), lambda i: (0, 0)),
            pl.BlockSpec((SED, 1), lambda i: (0, 0)),
            pl.BlockSpec((PED, 3 * SED), lambda i: (0, 0)),
            pl.BlockSpec((PED, 1), lambda i: (0, 0)),
        ],
        out_specs=pl.BlockSpec((PED, bs), lambda i: (0, i)),
        out_shape=jax.ShapeDtypeStruct((PED, B), jnp.float32),
    )(rows, tmT, tsT, mode_table, W_time, b_time2d, WpT, b_pref2d)


def kernel(user_id, transport_mode, timestamp, user_table, mode_table,
           W_time, b_time, W_pref, b_pref):
    info = plsc.get_sparse_core_info()
    NW = info.num_cores * info.num_subcores
    uid = user_id.astype(jnp.int32)
    idx2 = uid.reshape(NW, B // NW)
    rows = _sc_gather(user_table, idx2)
    outT = _tc_fused_t(
        rows,
        transport_mode.astype(jnp.int32).reshape(1, B),
        timestamp.T,
        mode_table,
        W_time,
        b_time.reshape(SED, 1),
        W_pref.T,
        b_pref.reshape(PED, 1),
    )
    return outT.T
